# baseline (speedup 1.0000x reference)
import functools

import jax
import jax.numpy as jnp
from jax import lax
from jax.experimental import pallas as pl
from jax.experimental.pallas import tpu as pltpu
from jax.experimental.pallas import tpu_sc as plsc

N = 10000
D = 128
S = 32
OUT = 128

NW = 32
NPW = 320
N_PAD = NW * NPW
G = 4
CHUNKS = NPW // G

D2 = D // 2


def _sc_maxpool(h_pk, idx3d):
    mesh = plsc.VectorSubcoreMesh(core_axis_name="c", subcore_axis_name="s")

    @functools.partial(
        pl.kernel,
        mesh=mesh,
        compiler_params=pltpu.CompilerParams(use_tc_tiling_on_sc=False),
        out_type=jax.ShapeDtypeStruct((NW, CHUNKS, G, D2), jnp.int32),
        scratch_types=[
            pltpu.VMEM((CHUNKS, G * S), jnp.int32),
            pltpu.VMEM((G * S,), jnp.int32),
            pltpu.VMEM((G * S,), jnp.int32),
            pltpu.VMEM((G * S, D2), jnp.int32),
            pltpu.VMEM((G * S, D2), jnp.int32),
            pltpu.VMEM((CHUNKS, G, D2), jnp.int32),
            pltpu.SemaphoreType.DMA,
            pltpu.SemaphoreType.DMA,
        ],
    )
    def pool(
        h_hbm, idx_hbm, out_hbm,
        idx_v, ib0, ib1, rows0, rows1, pool_v, sem0, sem1,
    ):
        sid = lax.axis_index("s")
        wid = sid * 2 + lax.axis_index("c")
        ibs = (ib0, ib1)
        rows = (rows0, rows1)
        sems = (sem0, sem1)

        pltpu.sync_copy(idx_hbm.at[wid], idx_v)

        def stage_idx(ci, b):
            for q in range(G * S // 16):
                sl = pl.ds(q * 16, 16)
                ibs[b][sl] = idx_v[ci, sl]

        def gather(b):
            return pltpu.make_async_copy(h_hbm.at[ibs[b]], rows[b], sems[b])

        stage_idx(0, 0)
        gather(0).start()
        stage_idx(1, 1)
        gather(1).start()

        sh16 = jnp.full((16,), 16, jnp.int32)
        mhi = jnp.full((16,), -0x10000, jnp.int32)

        def body(i, carry):
            for b in range(2):
                ci = i * 2 + b
                gather(b).wait()
                for g in range(G):
                    for c in range(D2 // 16):
                        sl = pl.ds(c * 16, 16)
                        w = rows[b][g * S, sl]
                        hi = w
                        lo = lax.shift_left(w, sh16)
                        for t in range(1, S):
                            w = rows[b][g * S + t, sl]
                            hi = jnp.maximum(hi, w)
                            lo = jnp.maximum(lo, lax.shift_left(w, sh16))
                        pool_v[ci, g, sl] = lax.bitwise_or(
                            lax.bitwise_and(hi, mhi),
                            lax.shift_right_logical(lo, sh16),
                        )
                nxt = ci + 2

                @pl.when(nxt < CHUNKS)
                def _():
                    stage_idx(nxt, b)
                    gather(b).start()

            return carry

        lax.fori_loop(0, CHUNKS // 2, body, 0)
        pltpu.sync_copy(pool_v, out_hbm.at[wid])

    return pool(h_pk, idx3d)


_BR = 400


def _tc_matmul(h, pooled, W):
    def body(h_ref, p_ref, w_ref, o_ref):
        o_ref[...] = jnp.dot(
            h_ref[...], w_ref[0:D, :], preferred_element_type=jnp.float32
        ) + jnp.dot(
            p_ref[...].astype(jnp.float32),
            w_ref[D : 2 * D, :],
            preferred_element_type=jnp.float32,
        )

    return pl.pallas_call(
        body,
        grid=(N // _BR,),
        in_specs=[
            pl.BlockSpec((_BR, D), lambda i: (i, 0)),
            pl.BlockSpec((_BR, D), lambda i: (i, 0)),
            pl.BlockSpec((2 * D, OUT), lambda i: (0, 0)),
        ],
        out_specs=pl.BlockSpec((_BR, OUT), lambda i: (i, 0)),
        out_shape=jax.ShapeDtypeStruct((N, OUT), jnp.float32),
    )(h, pooled, W)


def kernel(h, adj_list, aggregate_num, aggregate_neighbors, W):
    idx = jnp.pad(aggregate_neighbors, ((0, N_PAD - N), (0, 0)))
    u = lax.bitcast_convert_type(h.astype(jnp.bfloat16), jnp.uint16).astype(
        jnp.int32
    )
    key = lambda q: jnp.where(q >= 0x8000, q ^ 0x7FFF, q)
    s = key(u)
    h_pk = s[:, :D2] | (s[:, D2:] << 16)
    out_pk = _sc_maxpool(h_pk, idx.reshape(NW, CHUNKS, G * S))
    pk = out_pk.reshape(N_PAD, D2)[:N]
    lo = pk & 0xFFFF
    hi = (pk >> 16) & 0xFFFF
    u16 = jnp.concatenate([key(lo), key(hi)], axis=1).astype(jnp.uint16)
    pooled = lax.bitcast_convert_type(u16, jnp.bfloat16)
    return _tc_matmul(h, pooled, W)


# speedup vs baseline: 1.3710x; 1.3710x over previous
import functools

import jax
import jax.numpy as jnp
from jax import lax
from jax.experimental import pallas as pl
from jax.experimental.pallas import tpu as pltpu
from jax.experimental.pallas import tpu_sc as plsc

N = 10000
D = 128
S = 32
OUT = 128

NW = 32
NPW = 320
N_PAD = NW * NPW
G = 4
CHUNKS = NPW // G

D2 = D // 2


def _sc_maxpool(h_pk, idx3d):
    mesh = plsc.VectorSubcoreMesh(core_axis_name="c", subcore_axis_name="s")

    @functools.partial(
        pl.kernel,
        mesh=mesh,
        compiler_params=pltpu.CompilerParams(use_tc_tiling_on_sc=False),
        out_type=jax.ShapeDtypeStruct((NW, CHUNKS, G, D2), jnp.int32),
        scratch_types=[
            pltpu.VMEM((CHUNKS, G * S), jnp.int32),
            pltpu.VMEM((G * S,), jnp.int32),
            pltpu.VMEM((G * S,), jnp.int32),
            pltpu.VMEM((G * S, D2), jnp.int32),
            pltpu.VMEM((G * S, D2), jnp.int32),
            pltpu.VMEM((CHUNKS, G, D2), jnp.int32),
            pltpu.VMEM_SHARED((N_PAD, D2), jnp.int32),
            pltpu.SemaphoreType.DMA,
            pltpu.SemaphoreType.DMA,
        ],
    )
    def pool(
        h_hbm, idx_hbm, out_hbm,
        idx_v, ib0, ib1, rows0, rows1, pool_v, h_sp, sem0, sem1,
    ):
        sid = lax.axis_index("s")
        wid = sid * 2 + lax.axis_index("c")
        ibs = (ib0, ib1)
        rows = (rows0, rows1)
        sems = (sem0, sem1)

        seg = N_PAD // 16
        pltpu.sync_copy(
            h_hbm.at[pl.ds(sid * seg, seg)], h_sp.at[pl.ds(sid * seg, seg)]
        )
        pltpu.sync_copy(idx_hbm.at[wid], idx_v)
        plsc.subcore_barrier()

        def stage_idx(ci, b):
            for q in range(G * S // 16):
                sl = pl.ds(q * 16, 16)
                ibs[b][sl] = idx_v[ci, sl]

        def gather(b):
            return pltpu.make_async_copy(h_sp.at[ibs[b]], rows[b], sems[b])

        stage_idx(0, 0)
        gather(0).start()
        stage_idx(1, 1)
        gather(1).start()

        sh16 = jnp.full((16,), 16, jnp.int32)
        mhi = jnp.full((16,), -0x10000, jnp.int32)

        def body(i, carry):
            for b in range(2):
                ci = i * 2 + b
                gather(b).wait()
                for g in range(G):
                    for c in range(D2 // 16):
                        sl = pl.ds(c * 16, 16)
                        w = rows[b][g * S, sl]
                        hi = w
                        lo = lax.shift_left(w, sh16)
                        for t in range(1, S):
                            w = rows[b][g * S + t, sl]
                            hi = jnp.maximum(hi, w)
                            lo = jnp.maximum(lo, lax.shift_left(w, sh16))
                        pool_v[ci, g, sl] = lax.bitwise_or(
                            lax.bitwise_and(hi, mhi),
                            lax.shift_right_logical(lo, sh16),
                        )
                nxt = ci + 2

                @pl.when(nxt < CHUNKS)
                def _():
                    stage_idx(nxt, b)
                    gather(b).start()

            return carry

        lax.fori_loop(0, CHUNKS // 2, body, 0)
        pltpu.sync_copy(pool_v, out_hbm.at[wid])

    return pool(h_pk, idx3d)


_BR = 400


def _tc_matmul(h, pooled, W):
    def body(h_ref, p_ref, w_ref, o_ref):
        o_ref[...] = jnp.dot(
            h_ref[...], w_ref[0:D, :], preferred_element_type=jnp.float32
        ) + jnp.dot(
            p_ref[...].astype(jnp.float32),
            w_ref[D : 2 * D, :],
            preferred_element_type=jnp.float32,
        )

    return pl.pallas_call(
        body,
        grid=(N // _BR,),
        in_specs=[
            pl.BlockSpec((_BR, D), lambda i: (i, 0)),
            pl.BlockSpec((_BR, D), lambda i: (i, 0)),
            pl.BlockSpec((2 * D, OUT), lambda i: (0, 0)),
        ],
        out_specs=pl.BlockSpec((_BR, OUT), lambda i: (i, 0)),
        out_shape=jax.ShapeDtypeStruct((N, OUT), jnp.float32),
    )(h, pooled, W)


def kernel(h, adj_list, aggregate_num, aggregate_neighbors, W):
    idx = jnp.pad(aggregate_neighbors, ((0, N_PAD - N), (0, 0)))
    u = lax.bitcast_convert_type(h.astype(jnp.bfloat16), jnp.uint16).astype(
        jnp.int32
    )
    key = lambda q: jnp.where(q >= 0x8000, q ^ 0x7FFF, q)
    s = key(u)
    h_pk = jnp.pad(s[:, :D2] | (s[:, D2:] << 16), ((0, N_PAD - N), (0, 0)))
    out_pk = _sc_maxpool(h_pk, idx.reshape(NW, CHUNKS, G * S))
    pk = out_pk.reshape(N_PAD, D2)[:N]
    lo = pk & 0xFFFF
    hi = (pk >> 16) & 0xFFFF
    u16 = jnp.concatenate([key(lo), key(hi)], axis=1).astype(jnp.uint16)
    pooled = lax.bitcast_convert_type(u16, jnp.bfloat16)
    return _tc_matmul(h, pooled, W)


# speedup vs baseline: 2.2081x; 1.6105x over previous
import functools

import jax
import jax.numpy as jnp
from jax import lax
from jax.experimental import pallas as pl
from jax.experimental.pallas import tpu as pltpu
from jax.experimental.pallas import tpu_sc as plsc

N = 10000
D = 128
S = 32
OUT = 128

NW = 32
NPW = 320
N_PAD = NW * NPW
G = 4
CHUNKS = NPW // G


def _sc_maxpool(h_bf, idx3d):
    mesh = plsc.VectorSubcoreMesh(core_axis_name="c", subcore_axis_name="s")

    @functools.partial(
        pl.kernel,
        mesh=mesh,
        compiler_params=pltpu.CompilerParams(use_tc_tiling_on_sc=False),
        out_type=jax.ShapeDtypeStruct((NW, CHUNKS, G, D), jnp.bfloat16),
        scratch_types=[
            pltpu.VMEM((CHUNKS, G * S), jnp.int32),
            pltpu.VMEM((G * S,), jnp.int32),
            pltpu.VMEM((G * S,), jnp.int32),
            pltpu.VMEM((G * S, D), jnp.bfloat16),
            pltpu.VMEM((G * S, D), jnp.bfloat16),
            pltpu.VMEM((CHUNKS, G, D), jnp.bfloat16),
            pltpu.VMEM_SHARED((N_PAD, D), jnp.bfloat16),
            pltpu.SemaphoreType.DMA,
            pltpu.SemaphoreType.DMA,
        ],
    )
    def pool(
        h_hbm, idx_hbm, out_hbm,
        idx_v, ib0, ib1, rows0, rows1, pool_v, h_sp, sem0, sem1,
    ):
        sid = lax.axis_index("s")
        wid = sid * 2 + lax.axis_index("c")
        ibs = (ib0, ib1)
        rows = (rows0, rows1)
        sems = (sem0, sem1)

        seg = N_PAD // 16
        pltpu.sync_copy(
            h_hbm.at[pl.ds(sid * seg, seg)], h_sp.at[pl.ds(sid * seg, seg)]
        )
        pltpu.sync_copy(idx_hbm.at[wid], idx_v)
        plsc.subcore_barrier()

        def stage_idx(ci, b):
            for q in range(G * S // 16):
                sl = pl.ds(q * 16, 16)
                ibs[b][sl] = idx_v[ci, sl]

        def gather(b):
            return pltpu.make_async_copy(h_sp.at[ibs[b]], rows[b], sems[b])

        stage_idx(0, 0)
        gather(0).start()
        stage_idx(1, 1)
        gather(1).start()

        def body(i, carry):
            for b in range(2):
                ci = i * 2 + b
                gather(b).wait()
                for g in range(G):
                    for c in range(D // 32):
                        sl = pl.ds(c * 32, 32)
                        acc = rows[b][g * S, sl]
                        for t in range(1, S):
                            acc = jnp.maximum(acc, rows[b][g * S + t, sl])
                        pool_v[ci, g, sl] = acc
                nxt = ci + 2

                @pl.when(nxt < CHUNKS)
                def _():
                    stage_idx(nxt, b)
                    gather(b).start()

            return carry

        lax.fori_loop(0, CHUNKS // 2, body, 0)
        pltpu.sync_copy(pool_v, out_hbm.at[wid])

    return pool(h_bf, idx3d)


_BR = 400


def _tc_matmul(h, pooled, W):
    def body(h_ref, p_ref, w_ref, o_ref):
        o_ref[...] = jnp.dot(
            h_ref[...], w_ref[0:D, :], preferred_element_type=jnp.float32
        ) + jnp.dot(
            p_ref[...].astype(jnp.float32),
            w_ref[D : 2 * D, :],
            preferred_element_type=jnp.float32,
        )

    return pl.pallas_call(
        body,
        grid=(N // _BR,),
        in_specs=[
            pl.BlockSpec((_BR, D), lambda i: (i, 0)),
            pl.BlockSpec((_BR, D), lambda i: (i, 0)),
            pl.BlockSpec((2 * D, OUT), lambda i: (0, 0)),
        ],
        out_specs=pl.BlockSpec((_BR, OUT), lambda i: (i, 0)),
        out_shape=jax.ShapeDtypeStruct((N, OUT), jnp.float32),
    )(h, pooled, W)


def kernel(h, adj_list, aggregate_num, aggregate_neighbors, W):
    idx = jnp.pad(aggregate_neighbors, ((0, N_PAD - N), (0, 0)))
    h_bf = jnp.pad(h.astype(jnp.bfloat16), ((0, N_PAD - N), (0, 0)))
    out_bf = _sc_maxpool(h_bf, idx.reshape(NW, CHUNKS, G * S))
    pooled = out_bf.reshape(N_PAD, D)[:N]
    return _tc_matmul(h, pooled, W)
